# baseline (speedup 1.0000x reference)
import jax
import jax.numpy as jnp
from jax import lax
from jax.experimental import pallas as pl
from jax.experimental.pallas import tpu as pltpu
from jax.experimental.pallas import tpu_sc as plsc

B, V, F, E, C = 2, 10000, 128, 160000, 128


_MM_BM = 2000


def _mm_body(x_ref, w_ref, b_ref, h_ref, y_ref):
    hy = (
        jnp.dot(x_ref[...], w_ref[...], preferred_element_type=jnp.float32)
        + b_ref[...]
    )
    h_ref[...] = hy[:, :C]
    y_ref[...] = hy[:, C:]


def _input_matmul(Xf, Wcat, bcat):
    n = Xf.shape[0]
    return pl.pallas_call(
        _mm_body,
        grid=(n // _MM_BM,),
        in_specs=[
            pl.BlockSpec((_MM_BM, F), lambda i: (i, 0)),
            pl.BlockSpec((F, 2 * C), lambda i: (0, 0)),
            pl.BlockSpec((1, 2 * C), lambda i: (0, 0)),
        ],
        out_specs=[
            pl.BlockSpec((_MM_BM, C), lambda i: (i, 0)),
            pl.BlockSpec((_MM_BM, C), lambda i: (i, 0)),
        ],
        out_shape=[
            jax.ShapeDtypeStruct((n, C), jnp.float32),
            jax.ShapeDtypeStruct((n, C), jnp.float32),
        ],
    )(Xf, Wcat, bcat)


_EK = 80
_EPW = E // 16
_STEPS = _EPW // _EK
_VPW = 624
_VTAIL = V - 16 * _VPW


_SPW = 640
_STAIL = V - 15 * _SPW


def _sc_body(h_hbm, y_hbm, src_hbm, dst_hbm, attr_hbm,
             out_hbm, sout_hbm,
             acc_sh, s_sh, src_all, dst_all,
             rows_a, rows_b, attr_a, attr_b, sbuf_v,
             sem_ga, sem_gb, sem_ta, sem_tb,
             sem_sa, sem_sb, sem_ua, sem_ub):
    c = lax.axis_index("c")
    s = lax.axis_index("s")

    pltpu.sync_copy(h_hbm.at[c, pl.ds(s * _VPW, _VPW)],
                    acc_sh.at[pl.ds(s * _VPW, _VPW)])

    @pl.when(s == 15)
    def _():
        pltpu.sync_copy(h_hbm.at[c, pl.ds(16 * _VPW, _VTAIL)],
                        acc_sh.at[pl.ds(16 * _VPW, _VTAIL)])

    zeros16 = jnp.zeros((16,), jnp.float32)
    for k in range(_SPW // 16):
        sbuf_v[pl.ds(16 * k, 16)] = zeros16

    @pl.when(s < 15)
    def _():
        pltpu.sync_copy(sbuf_v, s_sh.at[pl.ds(s * _SPW, _SPW)])

    @pl.when(s == 15)
    def _():
        pltpu.sync_copy(sbuf_v.at[pl.ds(0, _STAIL)],
                        s_sh.at[pl.ds(15 * _SPW, _STAIL)])

    e_base = s * _EPW
    pltpu.sync_copy(src_hbm.at[pl.ds(e_base, _EPW)], src_all)
    pltpu.sync_copy(dst_hbm.at[pl.ds(e_base, _EPW)], dst_all)

    plsc.subcore_barrier()

    yc = y_hbm.at[c]
    a_base = c * E + e_base

    def gather(j, rows, attrb, sg, st):
        pltpu.async_copy(yc.at[src_all.at[pl.ds(j * _EK, _EK)]], rows, sg)
        pltpu.async_copy(attr_hbm.at[pl.ds(a_base + j * _EK, _EK)], attrb, st)

    def wait_gather(j, rows, attrb, sg, st):
        pltpu.make_async_copy(
            yc.at[src_all.at[pl.ds(j * _EK, _EK)]], rows, sg).wait()
        pltpu.make_async_copy(
            attr_hbm.at[pl.ds(a_base + j * _EK, _EK)], attrb, st).wait()

    def scatter(j, rows, attrb, ss, su):
        dchunk = dst_all.at[pl.ds(j * _EK, _EK)]
        pltpu.async_copy(rows, acc_sh.at[dchunk], ss, add=True)
        pltpu.async_copy(attrb, s_sh.at[dchunk], su, add=True)

    def wait_scatter(j, rows, attrb, ss, su):
        dchunk = dst_all.at[pl.ds(j * _EK, _EK)]
        pltpu.make_async_copy(rows, acc_sh.at[dchunk], ss).wait()
        pltpu.make_async_copy(attrb, s_sh.at[dchunk], su).wait()

    buf_a = (rows_a, attr_a)
    buf_b = (rows_b, attr_b)

    gather(0, *buf_a, sem_ga, sem_ta)
    gather(1, *buf_b, sem_gb, sem_tb)

    def step(k, carry):
        ja = 2 * k
        jb = ja + 1
        wait_gather(ja, *buf_a, sem_ga, sem_ta)
        scatter(ja, *buf_a, sem_sa, sem_ua)
        wait_gather(jb, *buf_b, sem_gb, sem_tb)
        scatter(jb, *buf_b, sem_sb, sem_ub)
        wait_scatter(ja, *buf_a, sem_sa, sem_ua)

        @pl.when(ja + 2 < _STEPS)
        def _():
            gather(ja + 2, *buf_a, sem_ga, sem_ta)

        wait_scatter(jb, *buf_b, sem_sb, sem_ub)

        @pl.when(jb + 2 < _STEPS)
        def _():
            gather(jb + 2, *buf_b, sem_gb, sem_tb)

        return carry

    lax.fori_loop(0, (_STEPS - 1) // 2, step, 0)
    wait_gather(_STEPS - 1, *buf_a, sem_ga, sem_ta)
    scatter(_STEPS - 1, *buf_a, sem_sa, sem_ua)
    wait_scatter(_STEPS - 1, *buf_a, sem_sa, sem_ua)
    plsc.subcore_barrier()

    pltpu.sync_copy(acc_sh.at[pl.ds(s * _VPW, _VPW)],
                    out_hbm.at[c, pl.ds(s * _VPW, _VPW)])

    @pl.when(s == 15)
    def _():
        pltpu.sync_copy(acc_sh.at[pl.ds(16 * _VPW, _VTAIL)],
                        out_hbm.at[c, pl.ds(16 * _VPW, _VTAIL)])

    @pl.when(s < 15)
    def _():
        pltpu.sync_copy(s_sh.at[pl.ds(s * _SPW, _SPW)], sbuf_v)
        pltpu.sync_copy(sbuf_v, sout_hbm.at[pl.ds(c * V + s * _SPW, _SPW)])

    @pl.when(s == 15)
    def _():
        pltpu.sync_copy(s_sh.at[pl.ds(15 * _SPW, _STAIL)],
                        sbuf_v.at[pl.ds(0, _STAIL)])
        pltpu.sync_copy(sbuf_v.at[pl.ds(0, _STAIL)],
                        sout_hbm.at[pl.ds(c * V + 15 * _SPW, _STAIL)])


_sc_scatter = pl.kernel(
    _sc_body,
    out_type=[
        jax.ShapeDtypeStruct((B, V, C), jnp.float32),
        jax.ShapeDtypeStruct((B * V,), jnp.float32),
    ],
    mesh=plsc.VectorSubcoreMesh(core_axis_name="c", subcore_axis_name="s"),
    scratch_types=[
        pltpu.VMEM_SHARED((V, C), jnp.float32),
        pltpu.VMEM_SHARED((V,), jnp.float32),
        pltpu.VMEM((_EPW,), jnp.int32),
        pltpu.VMEM((_EPW,), jnp.int32),
        pltpu.VMEM((_EK, C), jnp.float32),
        pltpu.VMEM((_EK, C), jnp.float32),
        pltpu.VMEM((_EK,), jnp.float32),
        pltpu.VMEM((_EK,), jnp.float32),
        pltpu.VMEM((_SPW,), jnp.float32),
        pltpu.SemaphoreType.DMA,
        pltpu.SemaphoreType.DMA,
        pltpu.SemaphoreType.DMA,
        pltpu.SemaphoreType.DMA,
        pltpu.SemaphoreType.DMA,
        pltpu.SemaphoreType.DMA,
        pltpu.SemaphoreType.DMA,
        pltpu.SemaphoreType.DMA,
    ],
)


_BN_BM = 2000
_BN_NB = (B * V) // _BN_BM


def _bn_body(hsum_ref, s_ref, wedge_ref, gamma_ref, beta_ref, out_ref, stat_ref):
    p = pl.program_id(0)
    i = pl.program_id(1)
    z = hsum_ref[...] + s_ref[...] * wedge_ref[...]

    @pl.when(p == 0)
    def _():
        @pl.when(i == 0)
        def _():
            stat_ref[0:2, :] = jnp.zeros((2, C), jnp.float32)

        stat_ref[0:1, :] += jnp.sum(z, axis=0, keepdims=True)
        stat_ref[1:2, :] += jnp.sum(z * z, axis=0, keepdims=True)

    @pl.when(p == 1)
    def _():
        @pl.when(i == 0)
        def _():
            n = jnp.float32(B * V)
            mean = stat_ref[0:1, :] / n
            var = stat_ref[1:2, :] / n - mean * mean
            scale = gamma_ref[...] * lax.rsqrt(var + 1e-5)
            stat_ref[2:3, :] = scale
            stat_ref[3:4, :] = beta_ref[...] - mean * scale

        out_ref[...] = jnp.maximum(z * stat_ref[2:3, :] + stat_ref[3:4, :], 0.0)


def _bn_relu(Hsum2, S2, wedge, gamma, beta):
    n = Hsum2.shape[0]
    return pl.pallas_call(
        _bn_body,
        grid=(2, _BN_NB),
        in_specs=[
            pl.BlockSpec((_BN_BM, C), lambda p, i: (i, 0)),
            pl.BlockSpec((_BN_BM, 1), lambda p, i: (i, 0)),
            pl.BlockSpec((1, C), lambda p, i: (0, 0)),
            pl.BlockSpec((1, C), lambda p, i: (0, 0)),
            pl.BlockSpec((1, C), lambda p, i: (0, 0)),
        ],
        out_specs=pl.BlockSpec((_BN_BM, C), lambda p, i: (i * p, 0)),
        out_shape=jax.ShapeDtypeStruct((n, C), jnp.float32),
        scratch_shapes=[pltpu.VMEM((4, C), jnp.float32)],
    )(Hsum2, S2, wedge, gamma, beta)


@jax.jit
def kernel(X, edge_index, edge_attr, W_node, W_edge, W_self, b_self, gamma, beta):
    src = edge_index[0].astype(jnp.int32)
    dst = edge_index[1].astype(jnp.int32)

    Xf = X.reshape(B * V, F)
    Wcat = jnp.concatenate([W_self, W_node], axis=1)
    bcat = jnp.concatenate([b_self, jnp.zeros((C,), jnp.float32)]).reshape(1, 2 * C)
    Hf, Yf = _input_matmul(Xf, Wcat, bcat)
    H = Hf.reshape(B, V, C)
    Y = Yf.reshape(B, V, C)

    Hsum, S = _sc_scatter(H, Y, src, dst, edge_attr.reshape(B * E))

    out = _bn_relu(Hsum.reshape(B * V, C), S.reshape(B * V, 1),
                   W_edge.reshape(1, C), gamma.reshape(1, C),
                   beta.reshape(1, C))
    return out.reshape(B, V, C)


# speedup vs baseline: 1.1679x; 1.1679x over previous
import jax
import jax.numpy as jnp
from jax import lax
from jax.experimental import pallas as pl
from jax.experimental.pallas import tpu as pltpu
from jax.experimental.pallas import tpu_sc as plsc

B, V, F, E, C = 2, 10000, 128, 160000, 128


_EK = 80
_EPW = E // 16
_STEPS = _EPW // _EK
_VPW = 624
_VTAIL = V - 16 * _VPW
_SPW = 640
_STAIL = V - 15 * _SPW


def _sc_body(x_hbm, src_hbm, dst_hbm, attr_hbm,
             out_hbm, sout_hbm,
             acc_sh, s_sh, src_all, dst_all,
             rows_a, rows_b, attr_a, attr_b, sbuf_v,
             sem_ga, sem_gb, sem_ta, sem_tb):
    c = lax.axis_index("c")
    s = lax.axis_index("s")

    zeros16 = jnp.zeros((16,), jnp.float32)

    def zrow(r, carry):
        for k in range(C // 16):
            rows_a[r, pl.ds(16 * k, 16)] = zeros16
        return carry

    lax.fori_loop(0, _EK, zrow, 0)
    for k in range(_SPW // 16):
        sbuf_v[pl.ds(16 * k, 16)] = zeros16

    v_base = s * _VPW
    for t in range(7):
        pltpu.sync_copy(rows_a, acc_sh.at[pl.ds(v_base + t * _EK, _EK)])
    pltpu.sync_copy(rows_a.at[pl.ds(0, _VPW - 7 * _EK)],
                    acc_sh.at[pl.ds(v_base + 7 * _EK, _VPW - 7 * _EK)])

    @pl.when(s == 15)
    def _():
        pltpu.sync_copy(rows_a.at[pl.ds(0, _VTAIL)],
                        acc_sh.at[pl.ds(16 * _VPW, _VTAIL)])

    @pl.when(s < 15)
    def _():
        pltpu.sync_copy(sbuf_v, s_sh.at[pl.ds(s * _SPW, _SPW)])

    @pl.when(s == 15)
    def _():
        pltpu.sync_copy(sbuf_v.at[pl.ds(0, _STAIL)],
                        s_sh.at[pl.ds(15 * _SPW, _STAIL)])

    e_base = s * _EPW
    pltpu.sync_copy(src_hbm.at[pl.ds(e_base, _EPW)], src_all)
    pltpu.sync_copy(dst_hbm.at[pl.ds(e_base, _EPW)], dst_all)

    plsc.subcore_barrier()

    xc = x_hbm.at[c]
    a_base = c * E + e_base

    def gather(j, rows, attrb, sg, st):
        pltpu.async_copy(xc.at[src_all.at[pl.ds(j * _EK, _EK)]], rows, sg)
        pltpu.async_copy(attr_hbm.at[pl.ds(a_base + j * _EK, _EK)], attrb, st)

    def drain(j, rows, attrb, sg, st):
        pltpu.make_async_copy(
            xc.at[src_all.at[pl.ds(j * _EK, _EK)]], rows, sg).wait()
        pltpu.make_async_copy(
            attr_hbm.at[pl.ds(a_base + j * _EK, _EK)], attrb, st).wait()
        dchunk = dst_all.at[pl.ds(j * _EK, _EK)]
        pltpu.sync_copy(rows, acc_sh.at[dchunk], add=True)
        pltpu.sync_copy(attrb, s_sh.at[dchunk], add=True)

    buf_a = (rows_a, attr_a)
    buf_b = (rows_b, attr_b)

    gather(0, *buf_a, sem_ga, sem_ta)

    def step(k, carry):
        ja = 2 * k
        gather(ja + 1, *buf_b, sem_gb, sem_tb)
        drain(ja, *buf_a, sem_ga, sem_ta)
        gather(ja + 2, *buf_a, sem_ga, sem_ta)
        drain(ja + 1, *buf_b, sem_gb, sem_tb)
        return carry

    lax.fori_loop(0, (_STEPS - 1) // 2, step, 0)
    drain(_STEPS - 1, *buf_a, sem_ga, sem_ta)
    plsc.subcore_barrier()

    pltpu.sync_copy(acc_sh.at[pl.ds(v_base, _VPW)],
                    out_hbm.at[c, pl.ds(v_base, _VPW)])

    @pl.when(s == 15)
    def _():
        pltpu.sync_copy(acc_sh.at[pl.ds(16 * _VPW, _VTAIL)],
                        out_hbm.at[c, pl.ds(16 * _VPW, _VTAIL)])

    @pl.when(s < 15)
    def _():
        pltpu.sync_copy(s_sh.at[pl.ds(s * _SPW, _SPW)], sbuf_v)
        pltpu.sync_copy(sbuf_v, sout_hbm.at[pl.ds(c * V + s * _SPW, _SPW)])

    @pl.when(s == 15)
    def _():
        pltpu.sync_copy(s_sh.at[pl.ds(15 * _SPW, _STAIL)],
                        sbuf_v.at[pl.ds(0, _STAIL)])
        pltpu.sync_copy(sbuf_v.at[pl.ds(0, _STAIL)],
                        sout_hbm.at[pl.ds(c * V + 15 * _SPW, _STAIL)])


_sc_scatter = pl.kernel(
    _sc_body,
    out_type=[
        jax.ShapeDtypeStruct((B, V, C), jnp.float32),
        jax.ShapeDtypeStruct((B * V,), jnp.float32),
    ],
    mesh=plsc.VectorSubcoreMesh(core_axis_name="c", subcore_axis_name="s"),
    scratch_types=[
        pltpu.VMEM_SHARED((V, C), jnp.float32),
        pltpu.VMEM_SHARED((V,), jnp.float32),
        pltpu.VMEM((_EPW,), jnp.int32),
        pltpu.VMEM((_EPW,), jnp.int32),
        pltpu.VMEM((_EK, C), jnp.float32),
        pltpu.VMEM((_EK, C), jnp.float32),
        pltpu.VMEM((_EK,), jnp.float32),
        pltpu.VMEM((_EK,), jnp.float32),
        pltpu.VMEM((_SPW,), jnp.float32),
        pltpu.SemaphoreType.DMA,
        pltpu.SemaphoreType.DMA,
        pltpu.SemaphoreType.DMA,
        pltpu.SemaphoreType.DMA,
    ],
)


_BN_BM = 2000
_BN_NB = (B * V) // _BN_BM


def _bn_body(x_ref, a_ref, s_ref, ws_ref, wn_ref, b_ref, wedge_ref,
             gamma_ref, beta_ref, out_ref, stat_ref):
    p = pl.program_id(0)
    i = pl.program_id(1)
    z = (
        jnp.dot(x_ref[...], ws_ref[...], preferred_element_type=jnp.float32)
        + jnp.dot(a_ref[...], wn_ref[...], preferred_element_type=jnp.float32)
        + b_ref[...]
        + s_ref[...] * wedge_ref[...]
    )

    @pl.when(p == 0)
    def _():
        @pl.when(i == 0)
        def _():
            stat_ref[0:2, :] = jnp.zeros((2, C), jnp.float32)

        stat_ref[0:1, :] += jnp.sum(z, axis=0, keepdims=True)
        stat_ref[1:2, :] += jnp.sum(z * z, axis=0, keepdims=True)

    @pl.when(p == 1)
    def _():
        @pl.when(i == 0)
        def _():
            n = jnp.float32(B * V)
            mean = stat_ref[0:1, :] / n
            var = stat_ref[1:2, :] / n - mean * mean
            scale = gamma_ref[...] * lax.rsqrt(var + 1e-5)
            stat_ref[2:3, :] = scale
            stat_ref[3:4, :] = beta_ref[...] - mean * scale

        out_ref[...] = jnp.maximum(z * stat_ref[2:3, :] + stat_ref[3:4, :], 0.0)


def _bn_relu(Xf, Af, S2, W_self, W_node, bias, wedge, gamma, beta):
    n = Xf.shape[0]
    return pl.pallas_call(
        _bn_body,
        grid=(2, _BN_NB),
        in_specs=[
            pl.BlockSpec((_BN_BM, F), lambda p, i: (i, 0)),
            pl.BlockSpec((_BN_BM, C), lambda p, i: (i, 0)),
            pl.BlockSpec((_BN_BM, 1), lambda p, i: (i, 0)),
            pl.BlockSpec((F, C), lambda p, i: (0, 0)),
            pl.BlockSpec((F, C), lambda p, i: (0, 0)),
            pl.BlockSpec((1, C), lambda p, i: (0, 0)),
            pl.BlockSpec((1, C), lambda p, i: (0, 0)),
            pl.BlockSpec((1, C), lambda p, i: (0, 0)),
            pl.BlockSpec((1, C), lambda p, i: (0, 0)),
        ],
        out_specs=pl.BlockSpec((_BN_BM, C), lambda p, i: (i * p, 0)),
        out_shape=jax.ShapeDtypeStruct((n, C), jnp.float32),
        scratch_shapes=[pltpu.VMEM((4, C), jnp.float32)],
    )(Xf, Af, S2, W_self, W_node, bias, wedge, gamma, beta)


@jax.jit
def kernel(X, edge_index, edge_attr, W_node, W_edge, W_self, b_self, gamma, beta):
    src = edge_index[0].astype(jnp.int32)
    dst = edge_index[1].astype(jnp.int32)

    accX, S = _sc_scatter(X, src, dst, edge_attr.reshape(B * E))

    out = _bn_relu(X.reshape(B * V, F), accX.reshape(B * V, C),
                   S.reshape(B * V, 1), W_self, W_node,
                   b_self.reshape(1, C), W_edge.reshape(1, C),
                   gamma.reshape(1, C), beta.reshape(1, C))
    return out.reshape(B, V, C)
